# baseline (speedup 1.0000x reference)
import functools

import jax
import jax.numpy as jnp
from jax import lax
from jax.experimental import pallas as pl
from jax.experimental.pallas import tpu as pltpu
from jax.experimental.pallas import tpu_sc as plsc

_B = 32
_N = 8732
_NP = 8736
_NC = _NP // 16
_NL = _N * 4
_NCL = _NL // 16
_NEGPOS = 3

_L1P = (
    0.0051860036,
    -0.029210268,
    0.07754038,
    -0.13583942,
    0.19055955,
    -0.24825649,
    0.3331601,
    -0.49999255,
    0.99999994,
)


def _log1p_exp_neg(ax):
    u = jnp.exp(-ax)
    q = jnp.full(u.shape, _L1P[0], jnp.float32)
    for c in _L1P[1:]:
        q = q * u + c
    return u * q


def _hist_select(hist_ref, nchunks, lane, t_count, k):
    lim = t_count - k

    def scan_body(c, carry):
        jcnt, csum = carry
        cnt = hist_ref[pl.ds(c * 16, 16)]
        cs = plsc.cumsum(cnt)
        prefix_excl = cs - cnt + csum
        flag = prefix_excl <= lim
        jcnt = jcnt + plsc.all_reduce_population_count(flag)
        return jcnt, csum + jnp.sum(cnt)

    jcnt, _ = lax.fori_loop(
        0, nchunks, scan_body, (jnp.zeros((16,), jnp.int32), jnp.int32(0))
    )
    jstar = jcnt - 1

    def pick_body(c, carry):
        above, h_at = carry
        cnt = hist_ref[pl.ds(c * 16, 16)]
        bid = lane + c * 16
        above = above + jnp.where(bid > jstar, cnt, 0)
        h_at = h_at + jnp.where(bid == jstar, cnt, 0)
        return above, h_at

    above, h_at = lax.fori_loop(
        0, nchunks, pick_body,
        (jnp.zeros((16,), jnp.int32), jnp.zeros((16,), jnp.int32)),
    )
    return jstar, jnp.sum(above), jnp.sum(h_at)


def _zero_hist(hist_ref, nchunks):
    def body(c, _):
        hist_ref[pl.ds(c * 16, 16)] = jnp.zeros((16,), jnp.int32)
        return 0

    lax.fori_loop(0, nchunks, body, 0)


def _sc_body(conf_hbm, ct_hbm, ld_hbm, lt_hbm, out_hbm,
             conf_v, ct_v, v_v, pf_v, ld_v, lt_v, hist, obuf,
             sem_c, sem_t, sem_l1, sem_l2):
    c_ax = lax.axis_index("c")
    s_ax = lax.axis_index("s")
    wid = s_ax * 2 + c_ax

    cp_c = pltpu.async_copy(conf_hbm.at[wid], conf_v, sem_c)
    cp_t = pltpu.async_copy(ct_hbm.at[wid], ct_v, sem_t)
    cp_l1 = pltpu.async_copy(ld_hbm.at[wid], ld_v, sem_l1)
    cp_l2 = pltpu.async_copy(lt_hbm.at[wid], lt_v, sem_l2)

    lane = jnp.arange(16, dtype=jnp.int32)
    _zero_hist(hist, 128)

    cp_c.wait()
    cp_t.wait()

    def p1_body(i, carry):
        npos_acc, bps_acc = carry
        o = i * 16
        x = conf_v[pl.ds(o, 16)]
        ct = ct_v[pl.ds(o, 16)]
        pos = ct > 0
        z = jnp.where(pos, 1.0, 0.0)
        bce = jnp.maximum(x, 0.0) - x * z + _log1p_exp_neg(jnp.abs(x))
        v = jnp.where(pos, 0.0, bce)
        v_v[pl.ds(o, 16)] = v
        pf_v[pl.ds(o, 16)] = z
        vb = plsc.bitcast(v, jnp.int32)
        plsc.addupdate_scatter(hist, [vb >> 20], jnp.full((16,), 1, jnp.int32))
        return npos_acc + z, bps_acc + bce * z

    npos_acc, bps_acc = lax.fori_loop(
        0, _NC, p1_body,
        (jnp.zeros((16,), jnp.float32), jnp.zeros((16,), jnp.float32)),
    )
    npos_f = jnp.sum(npos_acc)
    npos_i = npos_f.astype(jnp.int32)
    k = jnp.minimum(npos_i * _NEGPOS, _N)
    k0_vec = jnp.full((16,), 1, jnp.int32) * k
    k_vec = k0_vec

    t1 = jnp.full((16,), _NP, jnp.int32)
    b1, above1, h1 = _hist_select(hist, 128, lane, t1, k_vec)
    k_vec = k_vec - above1

    _zero_hist(hist, 64)

    def p2_body(i, _):
        o = i * 16
        vb = plsc.bitcast(v_v[pl.ds(o, 16)], jnp.int32)
        m = (vb >> 20) == b1
        plsc.addupdate_scatter(
            hist, [(vb >> 10) & 1023], jnp.full((16,), 1, jnp.int32), mask=m
        )
        return 0

    lax.fori_loop(0, _NC, p2_body, 0)
    t2 = jnp.full((16,), 1, jnp.int32) * h1
    b2, above2, h2 = _hist_select(hist, 64, lane, t2, k_vec)
    k_vec = k_vec - above2

    _zero_hist(hist, 64)
    b12 = (b1 << 10) | b2

    def p3_body(i, _):
        o = i * 16
        vb = plsc.bitcast(v_v[pl.ds(o, 16)], jnp.int32)
        m = (vb >> 10) == b12
        plsc.addupdate_scatter(
            hist, [vb & 1023], jnp.full((16,), 1, jnp.int32), mask=m
        )
        return 0

    lax.fori_loop(0, _NC, p3_body, 0)
    t3 = jnp.full((16,), 1, jnp.int32) * h2
    b3, _, _ = _hist_select(hist, 64, lane, t3, k_vec)

    t_vec = plsc.bitcast((b12 << 10) | b3, jnp.float32)

    def p4_body(i, carry):
        cgt_acc, s_acc = carry
        v = v_v[pl.ds(i * 16, 16)]
        gt = v > t_vec
        return (
            cgt_acc + jnp.where(gt, 1.0, 0.0),
            s_acc + jnp.where(gt, v, 0.0),
        )

    cgt_acc, s_acc = lax.fori_loop(
        0, _NC, p4_body,
        (jnp.zeros((16,), jnp.float32), jnp.zeros((16,), jnp.float32)),
    )

    cp_l1.wait()
    cp_l2.wait()
    lq = lane >> 2

    def p5_body(i, acc):
        o = i * 16
        a = ld_v[pl.ds(o, 16)]
        b = lt_v[pl.ds(o, 16)]
        d = a - b
        ad = jnp.abs(d)
        sl1 = jnp.where(ad < 1.0, (0.5 * d) * d, ad - 0.5)
        pf = plsc.load_gather(pf_v, [lq + i * 4])
        return acc + sl1 * pf

    sl1_acc = lax.fori_loop(0, _NCL, p5_body, jnp.zeros((16,), jnp.float32))

    cgt = jnp.sum(cgt_acc)
    s_gt = jnp.sum(s_acc)
    bps = jnp.sum(bps_acc)
    sl1s = jnp.sum(sl1_acc)
    kf = k0_vec.astype(jnp.float32)
    num_row = bps + s_gt + (kf - cgt) * t_vec
    den_row = npos_f + cgt + (kf - cgt) * jnp.where(t_vec > 0.0, 1.0, 0.0)

    part = jnp.where(
        lane == 0,
        sl1s,
        jnp.where(
            lane == 1,
            npos_f,
            jnp.where(lane == 2, num_row, jnp.where(lane == 3, den_row, 0.0)),
        ),
    )
    obuf[...] = part
    pltpu.sync_copy(obuf, out_hbm.at[wid])


_sc_kernel = functools.partial(
    pl.kernel,
    mesh=plsc.VectorSubcoreMesh(core_axis_name="c", subcore_axis_name="s"),
    out_type=jax.ShapeDtypeStruct((_B, 16), jnp.float32),
    compiler_params=pltpu.CompilerParams(needs_layout_passes=False),
    scratch_types=[
        pltpu.VMEM((_NP,), jnp.float32),
        pltpu.VMEM((_NP,), jnp.int32),
        pltpu.VMEM((_NP,), jnp.float32),
        pltpu.VMEM((_NP,), jnp.float32),
        pltpu.VMEM((_NL,), jnp.float32),
        pltpu.VMEM((_NL,), jnp.float32),
        pltpu.VMEM((2048,), jnp.int32),
        pltpu.VMEM((16,), jnp.float32),
        pltpu.SemaphoreType.DMA,
        pltpu.SemaphoreType.DMA,
        pltpu.SemaphoreType.DMA,
        pltpu.SemaphoreType.DMA,
    ],
)(_sc_body)


def _fin_body(p_ref, out_ref):
    p = p_ref[...]
    s = jnp.sum(p, axis=0, keepdims=True)
    col = lax.broadcasted_iota(jnp.int32, (1, 16), 1)
    S0 = jnp.sum(jnp.where(col == 0, s, 0.0))
    S1 = jnp.sum(jnp.where(col == 1, s, 0.0))
    S2 = jnp.sum(jnp.where(col == 2, s, 0.0))
    S3 = jnp.sum(jnp.where(col == 3, s, 0.0))
    loss_l = S0 / (4.0 * S1) / S1
    loss_c = S2 / S3 / S1
    col8 = lax.broadcasted_iota(jnp.int32, (1, 8), 1)
    out_ref[...] = jnp.where(col8 == 0, loss_l, jnp.where(col8 == 1, loss_c, 0.0))


def kernel(loc_data, conf_data, loc_t, conf_t):
    conf = conf_data[..., 0]
    conf_p = jnp.pad(conf, ((0, 0), (0, _NP - _N)), constant_values=-1e9)
    ct_p = jnp.pad(conf_t.astype(jnp.int32), ((0, 0), (0, _NP - _N)))
    ldf = loc_data.reshape(_B, _NL)
    ltf = loc_t.reshape(_B, _NL)

    partials = _sc_kernel(conf_p, ct_p, ldf, ltf)
    out = pl.pallas_call(
        _fin_body, out_shape=jax.ShapeDtypeStruct((1, 8), jnp.float32)
    )(partials)
    return (out[0, 0], out[0, 1])


# speedup vs baseline: 1.0797x; 1.0797x over previous
import functools

import jax
import jax.numpy as jnp
from jax import lax
from jax.experimental import pallas as pl
from jax.experimental.pallas import tpu as pltpu
from jax.experimental.pallas import tpu_sc as plsc

_B = 32
_N = 8732
_NP = 8736
_NC = _NP // 16
_NL = _N * 4
_NCL = _NL // 16
_NEGPOS = 3

_L1P = (
    0.0051860036,
    -0.029210268,
    0.07754038,
    -0.13583942,
    0.19055955,
    -0.24825649,
    0.3331601,
    -0.49999255,
    0.99999994,
)


def _log1p_exp_neg(ax):
    u = jnp.exp(-ax)
    q = jnp.full(u.shape, _L1P[0], jnp.float32)
    for c in _L1P[1:]:
        q = q * u + c
    return u * q


def _hist_select(hist_ref, nchunks, lane, t_count, k):
    lim = t_count - k

    def scan_body(c, carry):
        jcnt, csum = carry
        cnt = hist_ref[pl.ds(c * 16, 16)]
        cs = plsc.cumsum(cnt)
        prefix_excl = cs - cnt + csum
        flag = prefix_excl <= lim
        jcnt = jcnt + plsc.all_reduce_population_count(flag)
        return jcnt, csum + jnp.sum(cnt)

    jcnt, _ = lax.fori_loop(
        0, nchunks, scan_body, (jnp.zeros((16,), jnp.int32), jnp.int32(0)),
        unroll=2,
    )
    jstar = jcnt - 1

    def pick_body(c, carry):
        above, h_at = carry
        cnt = hist_ref[pl.ds(c * 16, 16)]
        bid = lane + c * 16
        above = above + jnp.where(bid > jstar, cnt, 0)
        h_at = h_at + jnp.where(bid == jstar, cnt, 0)
        return above, h_at

    above, h_at = lax.fori_loop(
        0, nchunks, pick_body,
        (jnp.zeros((16,), jnp.int32), jnp.zeros((16,), jnp.int32)),
        unroll=2,
    )
    return jstar, jnp.sum(above), jnp.sum(h_at)


def _zero_hist(hist_ref, nchunks):
    def body(c, _):
        hist_ref[pl.ds(c * 16, 16)] = jnp.zeros((16,), jnp.int32)
        return 0

    lax.fori_loop(0, nchunks, body, 0, unroll=4)


def _sc_body(conf_hbm, ct_hbm, ld_hbm, lt_hbm, out_hbm,
             conf_v, ct_v, v_v, pf_v, ld_v, lt_v, hist, obuf,
             sem_c, sem_t, sem_l1, sem_l2):
    c_ax = lax.axis_index("c")
    s_ax = lax.axis_index("s")
    wid = s_ax * 2 + c_ax

    cp_c = pltpu.async_copy(conf_hbm.at[wid], conf_v, sem_c)
    cp_t = pltpu.async_copy(ct_hbm.at[wid], ct_v, sem_t)
    cp_l1 = pltpu.async_copy(ld_hbm.at[wid], ld_v, sem_l1)
    cp_l2 = pltpu.async_copy(lt_hbm.at[wid], lt_v, sem_l2)

    lane = jnp.arange(16, dtype=jnp.int32)
    _zero_hist(hist, 128)

    cp_c.wait()
    cp_t.wait()

    ones_i = jnp.full((16,), 1, jnp.int32)

    def p1_body(i, carry):
        npos_acc, bps_acc, nz_acc = carry
        o = i * 16
        x = conf_v[pl.ds(o, 16)]
        ct = ct_v[pl.ds(o, 16)]
        pos = ct > 0
        z = jnp.where(pos, 1.0, 0.0)
        bce = jnp.maximum(x, 0.0) - x * z + _log1p_exp_neg(jnp.abs(x))
        v = jnp.where(pos, 0.0, bce)
        v_v[pl.ds(o, 16)] = v
        pf_v[pl.ds(o, 16)] = z
        vb = plsc.bitcast(v, jnp.int32)
        nzm = vb != 0
        plsc.addupdate_scatter(hist, [vb >> 20], ones_i, mask=nzm)
        nz_acc = nz_acc + plsc.all_reduce_population_count(nzm)
        return npos_acc + z, bps_acc + bce * z, nz_acc

    npos_acc, bps_acc, nz_acc = lax.fori_loop(
        0, _NC, p1_body,
        (jnp.zeros((16,), jnp.float32), jnp.zeros((16,), jnp.float32),
         jnp.zeros((16,), jnp.int32)),
        unroll=4,
    )
    z0 = _NP - jnp.sum(nz_acc) // 16
    h0 = hist[pl.ds(0, 16)]
    hist[pl.ds(0, 16)] = h0 + jnp.where(lane == 0, z0, 0)
    npos_f = jnp.sum(npos_acc)
    npos_i = npos_f.astype(jnp.int32)
    k = jnp.minimum(npos_i * _NEGPOS, _N)
    k0_vec = jnp.full((16,), 1, jnp.int32) * k
    k_vec = k0_vec

    t1 = jnp.full((16,), _NP, jnp.int32)
    b1, above1, h1 = _hist_select(hist, 128, lane, t1, k_vec)
    k_vec = k_vec - above1

    _zero_hist(hist, 64)

    def p2_body(i, _):
        o = i * 16
        vb = plsc.bitcast(v_v[pl.ds(o, 16)], jnp.int32)
        m = ((vb >> 20) == b1) & (vb != 0)
        plsc.addupdate_scatter(hist, [(vb >> 10) & 1023], ones_i, mask=m)
        return 0

    lax.fori_loop(0, _NC, p2_body, 0, unroll=4)
    h0 = hist[pl.ds(0, 16)]
    hist[pl.ds(0, 16)] = h0 + jnp.where((lane == 0) & (b1 == 0), z0, 0)
    t2 = jnp.full((16,), 1, jnp.int32) * h1
    b2, above2, h2 = _hist_select(hist, 64, lane, t2, k_vec)
    k_vec = k_vec - above2

    _zero_hist(hist, 64)
    b12 = (b1 << 10) | b2

    def p3_body(i, _):
        o = i * 16
        vb = plsc.bitcast(v_v[pl.ds(o, 16)], jnp.int32)
        m = ((vb >> 10) == b12) & (vb != 0)
        plsc.addupdate_scatter(hist, [vb & 1023], ones_i, mask=m)
        return 0

    lax.fori_loop(0, _NC, p3_body, 0, unroll=4)
    h0 = hist[pl.ds(0, 16)]
    hist[pl.ds(0, 16)] = h0 + jnp.where((lane == 0) & (b12 == 0), z0, 0)
    t3 = jnp.full((16,), 1, jnp.int32) * h2
    b3, _, _ = _hist_select(hist, 64, lane, t3, k_vec)

    t_vec = plsc.bitcast((b12 << 10) | b3, jnp.float32)

    def p4_body(i, carry):
        cgt_acc, s_acc = carry
        v = v_v[pl.ds(i * 16, 16)]
        gt = v > t_vec
        return (
            cgt_acc + jnp.where(gt, 1.0, 0.0),
            s_acc + jnp.where(gt, v, 0.0),
        )

    cgt_acc, s_acc = lax.fori_loop(
        0, _NC, p4_body,
        (jnp.zeros((16,), jnp.float32), jnp.zeros((16,), jnp.float32)),
        unroll=4,
    )

    cp_l1.wait()
    cp_l2.wait()
    lq = lane >> 2

    def p5_body(i, acc):
        o = i * 16
        a = ld_v[pl.ds(o, 16)]
        b = lt_v[pl.ds(o, 16)]
        d = a - b
        ad = jnp.abs(d)
        sl1 = jnp.where(ad < 1.0, (0.5 * d) * d, ad - 0.5)
        pf = plsc.load_gather(pf_v, [lq + i * 4])
        return acc + sl1 * pf

    sl1_acc = lax.fori_loop(
        0, _NCL, p5_body, jnp.zeros((16,), jnp.float32), unroll=4
    )

    cgt = jnp.sum(cgt_acc)
    s_gt = jnp.sum(s_acc)
    bps = jnp.sum(bps_acc)
    sl1s = jnp.sum(sl1_acc)
    kf = k0_vec.astype(jnp.float32)
    num_row = bps + s_gt + (kf - cgt) * t_vec
    den_row = npos_f + cgt + (kf - cgt) * jnp.where(t_vec > 0.0, 1.0, 0.0)

    part = jnp.where(
        lane == 0,
        sl1s,
        jnp.where(
            lane == 1,
            npos_f,
            jnp.where(lane == 2, num_row, jnp.where(lane == 3, den_row, 0.0)),
        ),
    )
    obuf[...] = part
    pltpu.sync_copy(obuf, out_hbm.at[wid])


_sc_kernel = functools.partial(
    pl.kernel,
    mesh=plsc.VectorSubcoreMesh(core_axis_name="c", subcore_axis_name="s"),
    out_type=jax.ShapeDtypeStruct((_B, 16), jnp.float32),
    compiler_params=pltpu.CompilerParams(needs_layout_passes=False),
    scratch_types=[
        pltpu.VMEM((_NP,), jnp.float32),
        pltpu.VMEM((_NP,), jnp.int32),
        pltpu.VMEM((_NP,), jnp.float32),
        pltpu.VMEM((_NP,), jnp.float32),
        pltpu.VMEM((_NL,), jnp.float32),
        pltpu.VMEM((_NL,), jnp.float32),
        pltpu.VMEM((2048,), jnp.int32),
        pltpu.VMEM((16,), jnp.float32),
        pltpu.SemaphoreType.DMA,
        pltpu.SemaphoreType.DMA,
        pltpu.SemaphoreType.DMA,
        pltpu.SemaphoreType.DMA,
    ],
)(_sc_body)


def _fin_body(p_ref, out_ref):
    p = p_ref[...]
    s = jnp.sum(p, axis=0, keepdims=True)
    col = lax.broadcasted_iota(jnp.int32, (1, 16), 1)
    S0 = jnp.sum(jnp.where(col == 0, s, 0.0))
    S1 = jnp.sum(jnp.where(col == 1, s, 0.0))
    S2 = jnp.sum(jnp.where(col == 2, s, 0.0))
    S3 = jnp.sum(jnp.where(col == 3, s, 0.0))
    loss_l = S0 / (4.0 * S1) / S1
    loss_c = S2 / S3 / S1
    col8 = lax.broadcasted_iota(jnp.int32, (1, 8), 1)
    out_ref[...] = jnp.where(col8 == 0, loss_l, jnp.where(col8 == 1, loss_c, 0.0))


def kernel(loc_data, conf_data, loc_t, conf_t):
    conf = conf_data[..., 0]
    conf_p = jnp.pad(conf, ((0, 0), (0, _NP - _N)), constant_values=-1e9)
    ct_p = jnp.pad(conf_t.astype(jnp.int32), ((0, 0), (0, _NP - _N)))
    ldf = loc_data.reshape(_B, _NL)
    ltf = loc_t.reshape(_B, _NL)

    partials = _sc_kernel(conf_p, ct_p, ldf, ltf)
    out = pl.pallas_call(
        _fin_body, out_shape=jax.ShapeDtypeStruct((1, 8), jnp.float32)
    )(partials)
    return (out[0, 0], out[0, 1])


# speedup vs baseline: 1.9261x; 1.7840x over previous
import functools

import jax
import jax.numpy as jnp
from jax import lax
from jax.experimental import pallas as pl
from jax.experimental.pallas import tpu as pltpu
from jax.experimental.pallas import tpu_sc as plsc

_B = 32
_N = 8732
_NP = 8736
_NC = _NP // 16
_NL = _N * 4
_NCL = _NL // 16
_NEGPOS = 3

_L1P = (
    0.0051860036,
    -0.029210268,
    0.07754038,
    -0.13583942,
    0.19055955,
    -0.24825649,
    0.3331601,
    -0.49999255,
    0.99999994,
)


def _log1p_exp_neg(ax):
    u = jnp.exp(-ax)
    q = jnp.full(u.shape, _L1P[0], jnp.float32)
    for c in _L1P[1:]:
        q = q * u + c
    return u * q


def _hist_select(hist_ref, nchunks, lane, t_count, k):
    lim = t_count - k

    def scan_body(c, carry):
        jcnt, csum = carry
        cnt = hist_ref[pl.ds(c * 16, 16)]
        cs = plsc.cumsum(cnt)
        prefix_excl = cs - cnt + csum
        flag = prefix_excl <= lim
        jcnt = jcnt + plsc.all_reduce_population_count(flag)
        return jcnt, csum + jnp.sum(cnt)

    jcnt, _ = lax.fori_loop(
        0, nchunks, scan_body, (jnp.zeros((16,), jnp.int32), jnp.int32(0)),
        unroll=2,
    )
    jstar = jcnt - 1

    def pick_body(c, carry):
        above, h_at = carry
        cnt = hist_ref[pl.ds(c * 16, 16)]
        bid = lane + c * 16
        above = above + jnp.where(bid > jstar, cnt, 0)
        h_at = h_at + jnp.where(bid == jstar, cnt, 0)
        return above, h_at

    above, h_at = lax.fori_loop(
        0, nchunks, pick_body,
        (jnp.zeros((16,), jnp.int32), jnp.zeros((16,), jnp.int32)),
        unroll=2,
    )
    return jstar, jnp.sum(above), jnp.sum(h_at)


def _zero_hist(hist_ref, nchunks):
    def body(c, _):
        hist_ref[pl.ds(c * 16, 16)] = jnp.zeros((16,), jnp.int32)
        return 0

    lax.fori_loop(0, nchunks, body, 0, unroll=4)


def _sc_body(conf_hbm, ct_hbm, out_hbm,
             conf_v, ct_v, v_v, hist, obuf,
             sem_c, sem_t):
    c_ax = lax.axis_index("c")
    s_ax = lax.axis_index("s")
    wid = s_ax * 2 + c_ax

    cp_c = pltpu.async_copy(conf_hbm.at[wid], conf_v, sem_c)
    cp_t = pltpu.async_copy(ct_hbm.at[wid], ct_v, sem_t)

    lane = jnp.arange(16, dtype=jnp.int32)
    _zero_hist(hist, 128)

    cp_c.wait()
    cp_t.wait()

    ones_i = jnp.full((16,), 1, jnp.int32)

    def p1_body(i, carry):
        npos_acc, bps_acc, nz_acc = carry
        o = i * 16
        x = conf_v[pl.ds(o, 16)]
        ct = ct_v[pl.ds(o, 16)]
        pos = ct > 0
        z = jnp.where(pos, 1.0, 0.0)
        bce = jnp.maximum(x, 0.0) - x * z + _log1p_exp_neg(jnp.abs(x))
        v = jnp.where(pos, 0.0, bce)
        v_v[pl.ds(o, 16)] = v
        vb = plsc.bitcast(v, jnp.int32)
        nzm = vb != 0
        plsc.addupdate_scatter(hist, [vb >> 20], ones_i, mask=nzm)
        nz_acc = nz_acc + plsc.all_reduce_population_count(nzm)
        return npos_acc + z, bps_acc + bce * z, nz_acc

    npos_acc, bps_acc, nz_acc = lax.fori_loop(
        0, _NC, p1_body,
        (jnp.zeros((16,), jnp.float32), jnp.zeros((16,), jnp.float32),
         jnp.zeros((16,), jnp.int32)),
        unroll=4,
    )
    z0 = _NP - jnp.sum(nz_acc) // 16
    h0 = hist[pl.ds(0, 16)]
    hist[pl.ds(0, 16)] = h0 + jnp.where(lane == 0, z0, 0)
    npos_f = jnp.sum(npos_acc)
    npos_i = npos_f.astype(jnp.int32)
    k = jnp.minimum(npos_i * _NEGPOS, _N)
    k0_vec = jnp.full((16,), 1, jnp.int32) * k
    k_vec = k0_vec

    t1 = jnp.full((16,), _NP, jnp.int32)
    b1, above1, h1 = _hist_select(hist, 128, lane, t1, k_vec)
    k_vec = k_vec - above1

    _zero_hist(hist, 64)

    def p2_body(i, _):
        o = i * 16
        vb = plsc.bitcast(v_v[pl.ds(o, 16)], jnp.int32)
        m = ((vb >> 20) == b1) & (vb != 0)
        plsc.addupdate_scatter(hist, [(vb >> 10) & 1023], ones_i, mask=m)
        return 0

    lax.fori_loop(0, _NC, p2_body, 0, unroll=4)
    h0 = hist[pl.ds(0, 16)]
    hist[pl.ds(0, 16)] = h0 + jnp.where((lane == 0) & (b1 == 0), z0, 0)
    t2 = jnp.full((16,), 1, jnp.int32) * h1
    b2, above2, h2 = _hist_select(hist, 64, lane, t2, k_vec)
    k_vec = k_vec - above2

    _zero_hist(hist, 64)
    b12 = (b1 << 10) | b2

    def p3_body(i, _):
        o = i * 16
        vb = plsc.bitcast(v_v[pl.ds(o, 16)], jnp.int32)
        m = ((vb >> 10) == b12) & (vb != 0)
        plsc.addupdate_scatter(hist, [vb & 1023], ones_i, mask=m)
        return 0

    lax.fori_loop(0, _NC, p3_body, 0, unroll=4)
    h0 = hist[pl.ds(0, 16)]
    hist[pl.ds(0, 16)] = h0 + jnp.where((lane == 0) & (b12 == 0), z0, 0)
    t3 = jnp.full((16,), 1, jnp.int32) * h2
    b3, _, _ = _hist_select(hist, 64, lane, t3, k_vec)

    t_vec = plsc.bitcast((b12 << 10) | b3, jnp.float32)

    def p4_body(i, carry):
        cgt_acc, s_acc = carry
        v = v_v[pl.ds(i * 16, 16)]
        gt = v > t_vec
        return (
            cgt_acc + jnp.where(gt, 1.0, 0.0),
            s_acc + jnp.where(gt, v, 0.0),
        )

    cgt_acc, s_acc = lax.fori_loop(
        0, _NC, p4_body,
        (jnp.zeros((16,), jnp.float32), jnp.zeros((16,), jnp.float32)),
        unroll=4,
    )

    cgt = jnp.sum(cgt_acc)
    s_gt = jnp.sum(s_acc)
    bps = jnp.sum(bps_acc)
    sl1s = jnp.float32(0.0)
    kf = k0_vec.astype(jnp.float32)
    num_row = bps + s_gt + (kf - cgt) * t_vec
    den_row = npos_f + cgt + (kf - cgt) * jnp.where(t_vec > 0.0, 1.0, 0.0)

    part = jnp.where(
        lane == 0,
        sl1s,
        jnp.where(
            lane == 1,
            npos_f,
            jnp.where(lane == 2, num_row, jnp.where(lane == 3, den_row, 0.0)),
        ),
    )
    obuf[...] = part
    pltpu.sync_copy(obuf, out_hbm.at[wid])


_sc_kernel = functools.partial(
    pl.kernel,
    mesh=plsc.VectorSubcoreMesh(core_axis_name="c", subcore_axis_name="s"),
    out_type=jax.ShapeDtypeStruct((_B, 16), jnp.float32),
    compiler_params=pltpu.CompilerParams(needs_layout_passes=False),
    scratch_types=[
        pltpu.VMEM((_NP,), jnp.float32),
        pltpu.VMEM((_NP,), jnp.int32),
        pltpu.VMEM((_NP,), jnp.float32),
        pltpu.VMEM((2048,), jnp.int32),
        pltpu.VMEM((16,), jnp.float32),
        pltpu.SemaphoreType.DMA,
        pltpu.SemaphoreType.DMA,
    ],
)(_sc_body)


def _fin_body(p_ref, ld_ref, lt_ref, ct_ref, out_ref):
    posf = (ct_ref[...] > 0).astype(jnp.float32)
    sl1_box = jnp.zeros(posf.shape, jnp.float32)
    for c in range(4):
        d = ld_ref[c] - lt_ref[c]
        ad = jnp.abs(d)
        sl1_box = sl1_box + jnp.where(ad < 1.0, (0.5 * d) * d, ad - 0.5)
    S0 = jnp.sum(sl1_box * posf)

    p = p_ref[...]
    s = jnp.sum(p, axis=0, keepdims=True)
    col = lax.broadcasted_iota(jnp.int32, (1, 16), 1)
    S1 = jnp.sum(jnp.where(col == 1, s, 0.0))
    S2 = jnp.sum(jnp.where(col == 2, s, 0.0))
    S3 = jnp.sum(jnp.where(col == 3, s, 0.0))
    loss_l = S0 / (4.0 * S1) / S1
    loss_c = S2 / S3 / S1
    col8 = lax.broadcasted_iota(jnp.int32, (1, 8), 1)
    out_ref[...] = jnp.where(col8 == 0, loss_l, jnp.where(col8 == 1, loss_c, 0.0))


def kernel(loc_data, conf_data, loc_t, conf_t):
    conf = conf_data[..., 0]
    conf_p = jnp.pad(conf, ((0, 0), (0, _NP - _N)), constant_values=-1e9)
    ct = conf_t.astype(jnp.int32)
    ct_p = jnp.pad(ct, ((0, 0), (0, _NP - _N)))
    ldT = jnp.transpose(loc_data, (2, 0, 1))
    ltT = jnp.transpose(loc_t, (2, 0, 1))

    partials = _sc_kernel(conf_p, ct_p)
    out = pl.pallas_call(
        _fin_body, out_shape=jax.ShapeDtypeStruct((1, 8), jnp.float32)
    )(partials, ldT, ltT, ct)
    return (out[0, 0], out[0, 1])


# speedup vs baseline: 3.0723x; 1.5951x over previous
import functools

import jax
import jax.numpy as jnp
from jax import lax
from jax.experimental import pallas as pl
from jax.experimental.pallas import tpu as pltpu
from jax.experimental.pallas import tpu_sc as plsc

_B = 32
_N = 8732
_NP = 8736
_NC = _NP // 16
_NL = _N * 4
_NCL = _NL // 16
_NEGPOS = 3

_L1P = (
    0.0051860036,
    -0.029210268,
    0.07754038,
    -0.13583942,
    0.19055955,
    -0.24825649,
    0.3331601,
    -0.49999255,
    0.99999994,
)


def _log1p_exp_neg(ax):
    u = jnp.exp(-ax)
    q = jnp.full(u.shape, _L1P[0], jnp.float32)
    for c in _L1P[1:]:
        q = q * u + c
    return u * q


def _hist_select(hist_ref, nchunks, lane, t_count, k):
    lim = t_count - k

    @plsc.parallel_loop(
        0, nchunks * 16, 16, unroll=2,
        carry=(jnp.zeros((16,), jnp.int32), jnp.int32(0)),
    )
    def scan_carry(o, carry):
        jcnt, csum = carry
        cnt = hist_ref[pl.ds(o, 16)]
        cs = plsc.cumsum(cnt)
        prefix_excl = cs - cnt + csum
        flag = prefix_excl <= lim
        jcnt = jcnt + plsc.all_reduce_population_count(flag)
        return jcnt, csum + cs[15]

    jcnt, _ = scan_carry
    jstar = jcnt - 1

    @plsc.parallel_loop(
        0, nchunks * 16, 16, unroll=2,
        carry=(jnp.zeros((16,), jnp.int32), jnp.zeros((16,), jnp.int32)),
    )
    def pick_carry(o, carry):
        above, h_at = carry
        cnt = hist_ref[pl.ds(o, 16)]
        bid = lane + o
        above = above + jnp.where(bid > jstar, cnt, 0)
        h_at = h_at + jnp.where(bid == jstar, cnt, 0)
        return above, h_at

    above, h_at = pick_carry
    return jstar, jnp.sum(above), jnp.sum(h_at)


def _zero_hist(hist_ref, nchunks):
    @plsc.parallel_loop(0, nchunks * 16, 16, unroll=4)
    def zero_loop(o):
        hist_ref[pl.ds(o, 16)] = jnp.zeros((16,), jnp.int32)


def _sc_body(conf_hbm, ct_hbm, out_hbm,
             conf_v, ct_v, v_v, hist, obuf,
             sem_c, sem_t):
    c_ax = lax.axis_index("c")
    s_ax = lax.axis_index("s")
    wid = s_ax * 2 + c_ax

    cp_c = pltpu.async_copy(conf_hbm.at[wid], conf_v, sem_c)
    cp_t = pltpu.async_copy(ct_hbm.at[wid], ct_v, sem_t)

    lane = jnp.arange(16, dtype=jnp.int32)
    _zero_hist(hist, 128)

    cp_c.wait()
    cp_t.wait()

    ones_i = jnp.full((16,), 1, jnp.int32)

    @plsc.parallel_loop(
        0, _NP, 16, unroll=4,
        carry=(jnp.zeros((16,), jnp.float32), jnp.zeros((16,), jnp.float32),
               jnp.zeros((16,), jnp.int32)),
    )
    def p1_carry(o, carry):
        npos_acc, bps_acc, nz_acc = carry
        x = conf_v[pl.ds(o, 16)]
        ct = ct_v[pl.ds(o, 16)]
        pos = ct > 0
        z = jnp.where(pos, 1.0, 0.0)
        bce = jnp.maximum(x, 0.0) - x * z + _log1p_exp_neg(jnp.abs(x))
        v = jnp.where(pos, 0.0, bce)
        v_v[pl.ds(o, 16)] = v
        vb = plsc.bitcast(v, jnp.int32)
        nzm = vb != 0
        plsc.addupdate_scatter(hist, [vb >> 20], ones_i, mask=nzm)
        nz_acc = nz_acc + plsc.all_reduce_population_count(nzm)
        return npos_acc + z, bps_acc + bce * z, nz_acc

    npos_acc, bps_acc, nz_acc = p1_carry
    z0 = _NP - jnp.sum(nz_acc) // 16
    h0 = hist[pl.ds(0, 16)]
    hist[pl.ds(0, 16)] = h0 + jnp.where(lane == 0, z0, 0)
    npos_f = jnp.sum(npos_acc)
    npos_i = npos_f.astype(jnp.int32)
    k = jnp.minimum(npos_i * _NEGPOS, _N)
    k0_vec = jnp.full((16,), 1, jnp.int32) * k
    k_vec = k0_vec

    t1 = jnp.full((16,), _NP, jnp.int32)
    b1, above1, h1 = _hist_select(hist, 128, lane, t1, k_vec)
    k_vec = k_vec - above1

    _zero_hist(hist, 64)

    @plsc.parallel_loop(0, _NP, 16, unroll=4)
    def p2_loop(o):
        vb = plsc.bitcast(v_v[pl.ds(o, 16)], jnp.int32)
        m = ((vb >> 20) == b1) & (vb != 0)
        plsc.addupdate_scatter(hist, [(vb >> 10) & 1023], ones_i, mask=m)
    h0 = hist[pl.ds(0, 16)]
    hist[pl.ds(0, 16)] = h0 + jnp.where((lane == 0) & (b1 == 0), z0, 0)
    t2 = jnp.full((16,), 1, jnp.int32) * h1
    b2, above2, h2 = _hist_select(hist, 64, lane, t2, k_vec)
    k_vec = k_vec - above2

    _zero_hist(hist, 64)
    b12 = (b1 << 10) | b2

    @plsc.parallel_loop(0, _NP, 16, unroll=4)
    def p3_loop(o):
        vb = plsc.bitcast(v_v[pl.ds(o, 16)], jnp.int32)
        m = ((vb >> 10) == b12) & (vb != 0)
        plsc.addupdate_scatter(hist, [vb & 1023], ones_i, mask=m)
    h0 = hist[pl.ds(0, 16)]
    hist[pl.ds(0, 16)] = h0 + jnp.where((lane == 0) & (b12 == 0), z0, 0)
    t3 = jnp.full((16,), 1, jnp.int32) * h2
    b3, _, _ = _hist_select(hist, 64, lane, t3, k_vec)

    t_vec = plsc.bitcast((b12 << 10) | b3, jnp.float32)

    @plsc.parallel_loop(
        0, _NP, 16, unroll=4,
        carry=(jnp.zeros((16,), jnp.float32), jnp.zeros((16,), jnp.float32)),
    )
    def p4_carry(o, carry):
        cgt_acc, s_acc = carry
        v = v_v[pl.ds(o, 16)]
        gt = v > t_vec
        return (
            cgt_acc + jnp.where(gt, 1.0, 0.0),
            s_acc + jnp.where(gt, v, 0.0),
        )

    cgt_acc, s_acc = p4_carry

    cgt = jnp.sum(cgt_acc)
    s_gt = jnp.sum(s_acc)
    bps = jnp.sum(bps_acc)
    sl1s = jnp.float32(0.0)
    kf = k0_vec.astype(jnp.float32)
    num_row = bps + s_gt + (kf - cgt) * t_vec
    den_row = npos_f + cgt + (kf - cgt) * jnp.where(t_vec > 0.0, 1.0, 0.0)

    part = jnp.where(
        lane == 0,
        sl1s,
        jnp.where(
            lane == 1,
            npos_f,
            jnp.where(lane == 2, num_row, jnp.where(lane == 3, den_row, 0.0)),
        ),
    )
    obuf[...] = part
    pltpu.sync_copy(obuf, out_hbm.at[wid])


_sc_kernel = functools.partial(
    pl.kernel,
    mesh=plsc.VectorSubcoreMesh(core_axis_name="c", subcore_axis_name="s"),
    out_type=jax.ShapeDtypeStruct((_B, 16), jnp.float32),
    compiler_params=pltpu.CompilerParams(needs_layout_passes=False),
    scratch_types=[
        pltpu.VMEM((_NP,), jnp.float32),
        pltpu.VMEM((_NP,), jnp.int32),
        pltpu.VMEM((_NP,), jnp.float32),
        pltpu.VMEM((2048,), jnp.int32),
        pltpu.VMEM((16,), jnp.float32),
        pltpu.SemaphoreType.DMA,
        pltpu.SemaphoreType.DMA,
    ],
)(_sc_body)


def _fin_body(p_ref, ld_ref, lt_ref, ct_ref, out_ref):
    posf = (ct_ref[...] > 0).astype(jnp.float32)
    sl1_box = jnp.zeros(posf.shape, jnp.float32)
    for c in range(4):
        d = ld_ref[c] - lt_ref[c]
        ad = jnp.abs(d)
        sl1_box = sl1_box + jnp.where(ad < 1.0, (0.5 * d) * d, ad - 0.5)
    S0 = jnp.sum(sl1_box * posf)

    p = p_ref[...]
    s = jnp.sum(p, axis=0, keepdims=True)
    col = lax.broadcasted_iota(jnp.int32, (1, 16), 1)
    S1 = jnp.sum(jnp.where(col == 1, s, 0.0))
    S2 = jnp.sum(jnp.where(col == 2, s, 0.0))
    S3 = jnp.sum(jnp.where(col == 3, s, 0.0))
    loss_l = S0 / (4.0 * S1) / S1
    loss_c = S2 / S3 / S1
    col8 = lax.broadcasted_iota(jnp.int32, (1, 8), 1)
    out_ref[...] = jnp.where(col8 == 0, loss_l, jnp.where(col8 == 1, loss_c, 0.0))


def kernel(loc_data, conf_data, loc_t, conf_t):
    conf = conf_data[..., 0]
    conf_p = jnp.pad(conf, ((0, 0), (0, _NP - _N)), constant_values=-1e9)
    ct = conf_t.astype(jnp.int32)
    ct_p = jnp.pad(ct, ((0, 0), (0, _NP - _N)))
    ldT = jnp.transpose(loc_data, (2, 0, 1))
    ltT = jnp.transpose(loc_t, (2, 0, 1))

    partials = _sc_kernel(conf_p, ct_p)
    out = pl.pallas_call(
        _fin_body, out_shape=jax.ShapeDtypeStruct((1, 8), jnp.float32)
    )(partials, ldT, ltT, ct)
    return (out[0, 0], out[0, 1])
